# baseline (speedup 1.0000x reference)
import functools

import jax
import jax.numpy as jnp
from jax import lax
from jax.experimental import pallas as pl
from jax.experimental.pallas import tpu as pltpu
from jax.experimental.pallas import tpu_sc as plsc

NUM_EMB = 1_000_000
DIM = 64
SEQS = 16384
SLEN = 50
BATCH = SEQS * SLEN

_info = plsc.get_sparse_core_info()
_NC, _NS = _info.num_cores, _info.num_subcores
NW = _NC * _NS
ROWS_PER_W = BATCH // NW
GCH = 128
K = 2
SCH = K * GCH
S = ROWS_PER_W // SCH
G_PER_W = ROWS_PER_W // GCH
NBUF = 3

_mesh = plsc.VectorSubcoreMesh(core_axis_name="c", subcore_axis_name="s")

PRE_BLK = 2000


def _pad_body(w_ref, o_ref):
    o_ref[...] = jnp.concatenate(
        [w_ref[...], jnp.zeros((PRE_BLK, DIM), jnp.float32)], axis=1)


_pad_table = pl.pallas_call(
    _pad_body,
    grid=(NUM_EMB // PRE_BLK,),
    in_specs=[pl.BlockSpec((PRE_BLK, DIM), lambda i: (i, 0))],
    out_specs=pl.BlockSpec((PRE_BLK, 2 * DIM), lambda i: (i, 0)),
    out_shape=jax.ShapeDtypeStruct((NUM_EMB, 2 * DIM), jnp.float32),
)

POST_SEQ = 64


def _fmt_body(x_ref, o_ref):
    x = x_ref[...]
    o_ref[...] = x[:, :DIM].reshape(POST_SEQ, SLEN, DIM)


_fmt_out = pl.pallas_call(
    _fmt_body,
    grid=(SEQS // POST_SEQ,),
    in_specs=[pl.BlockSpec((POST_SEQ * SLEN, 2 * DIM), lambda i: (i, 0))],
    out_specs=pl.BlockSpec((POST_SEQ, SLEN, DIM), lambda i: (i, 0, 0)),
    out_shape=jax.ShapeDtypeStruct((SEQS, SLEN, DIM), jnp.float32),
)


@functools.partial(
    pl.kernel,
    mesh=_mesh,
    out_type=jax.ShapeDtypeStruct((BATCH, 2 * DIM), jnp.float32),
    scratch_types=[
        pltpu.VMEM((G_PER_W, GCH), jnp.int32),
        pltpu.VMEM((SCH, 2 * DIM), jnp.float32),
        pltpu.VMEM((SCH, 2 * DIM), jnp.float32),
        pltpu.VMEM((SCH, 2 * DIM), jnp.float32),
        pltpu.SemaphoreType.DMA,
        pltpu.SemaphoreType.DMA,
        pltpu.SemaphoreType.DMA,
        pltpu.SemaphoreType.DMA,
    ],
    compiler_params=pltpu.CompilerParams(use_tc_tiling_on_sc=False),
)
def _emb_lookup(idx_hbm, table_hbm, out_hbm, idx_v, buf0, buf1, buf2,
                sem_g, sem0, sem1, sem2):
    bufs = (buf0, buf1, buf2)
    sems = (sem0, sem1, sem2)
    wid = lax.axis_index("s") * _NC + lax.axis_index("c")
    gbase = wid * G_PER_W
    rbase = wid * ROWS_PER_W
    pltpu.sync_copy(idx_hbm.at[pl.ds(gbase, G_PER_W)], idx_v)

    def fire_gathers(s, b):
        for k in range(K):
            pltpu.async_copy(
                table_hbm.at[idx_v.at[s * K + k]],
                bufs[b].at[pl.ds(k * GCH, GCH)],
                sem_g,
            )

    def wait_gathers(b):
        pltpu.make_async_copy(
            table_hbm.at[pl.ds(0, SCH)], bufs[b], sem_g).wait()

    def fire_out(s, b):
        pltpu.async_copy(
            bufs[b], out_hbm.at[pl.ds(rbase + s * SCH, SCH)], sems[b])

    def wait_out(b):
        pltpu.make_async_copy(
            bufs[b], out_hbm.at[pl.ds(rbase, SCH)], sems[b]).wait()

    def step(s, b, fire_ahead=True):
        wait_gathers(b)
        fire_out(s, b)
        if fire_ahead:
            nb = (b + 2) % NBUF
            wait_out(nb)
            fire_gathers(s + 2, nb)

    fire_gathers(0, 0)
    fire_gathers(1, 1)
    wait_gathers(0)
    fire_out(0, 0)
    fire_gathers(2, 2)

    def body(t, carry):
        s = 3 * t + 1
        step(s, 1)
        step(s + 1, 2)
        step(s + 2, 0)
        return carry

    T = (S - 7) // 3
    lax.fori_loop(0, T, body, 0)
    for s in range(3 * T + 1, S):
        step(s, s % NBUF, fire_ahead=(s + 2 < S))
    wait_out((S - 3) % NBUF)
    wait_out((S - 2) % NBUF)
    wait_out((S - 1) % NBUF)


def kernel(token_ids, weight):
    idx = token_ids.reshape(NW * G_PER_W, GCH).astype(jnp.int32)
    wpad = _pad_table(weight)
    rows = _emb_lookup(idx, wpad)
    return _fmt_out(rows)


# speedup vs baseline: 1.2667x; 1.2667x over previous
import functools

import jax
import jax.numpy as jnp
from jax import lax
from jax.experimental import pallas as pl
from jax.experimental.pallas import tpu as pltpu
from jax.experimental.pallas import tpu_sc as plsc

NUM_EMB = 1_000_000
DIM = 64
SEQS = 16384
SLEN = 50
BATCH = SEQS * SLEN

_info = plsc.get_sparse_core_info()
_NC, _NS = _info.num_cores, _info.num_subcores
NW = _NC * _NS
ROWS_PER_W = BATCH // NW
GCH = 128
K = 2
SCH = K * GCH
S = ROWS_PER_W // SCH
G_PER_W = ROWS_PER_W // GCH
NBUF = 3

_mesh = plsc.VectorSubcoreMesh(core_axis_name="c", subcore_axis_name="s")

PRE_BLK = 2000


def _pad_body(w_ref, o_ref):
    o_ref[...] = jnp.concatenate(
        [w_ref[...], jnp.zeros((PRE_BLK, DIM), jnp.float32)], axis=1)


_pad_table = pl.pallas_call(
    _pad_body,
    grid=(NUM_EMB // PRE_BLK,),
    in_specs=[pl.BlockSpec((PRE_BLK, DIM), lambda i: (i, 0))],
    out_specs=pl.BlockSpec((PRE_BLK, 2 * DIM), lambda i: (i, 0)),
    out_shape=jax.ShapeDtypeStruct((NUM_EMB, 2 * DIM), jnp.float32),
)

POST_SEQ = 64


def _fmt_body(x_ref, o_ref):
    x = x_ref[...]
    o_ref[...] = x[:, :DIM].reshape(POST_SEQ, SLEN, DIM)


_fmt_out = pl.pallas_call(
    _fmt_body,
    grid=(SEQS // POST_SEQ,),
    in_specs=[pl.BlockSpec((POST_SEQ * SLEN, 2 * DIM), lambda i: (i, 0))],
    out_specs=pl.BlockSpec((POST_SEQ, SLEN, DIM), lambda i: (i, 0, 0)),
    out_shape=jax.ShapeDtypeStruct((SEQS, SLEN, DIM), jnp.float32),
)


@functools.partial(
    pl.kernel,
    mesh=_mesh,
    out_type=jax.ShapeDtypeStruct((BATCH, 2 * DIM), jnp.float32),
    scratch_types=[
        pltpu.VMEM((G_PER_W, GCH), jnp.int32),
        pltpu.VMEM((SCH, 2 * DIM), jnp.float32),
        pltpu.VMEM((SCH, 2 * DIM), jnp.float32),
        pltpu.VMEM((SCH, 2 * DIM), jnp.float32),
        pltpu.SemaphoreType.DMA,
        pltpu.SemaphoreType.DMA,
        pltpu.SemaphoreType.DMA,
        pltpu.SemaphoreType.DMA,
    ],
    compiler_params=pltpu.CompilerParams(use_tc_tiling_on_sc=False),
)
def _emb_lookup(idx_hbm, table_hbm, out_hbm, idx_v, buf0, buf1, buf2,
                sem_g, sem0, sem1, sem2):
    bufs = (buf0, buf1, buf2)
    sems = (sem0, sem1, sem2)
    wid = lax.axis_index("s") * _NC + lax.axis_index("c")
    gbase = wid * G_PER_W
    rbase = wid * ROWS_PER_W
    pltpu.sync_copy(idx_hbm.at[pl.ds(gbase, G_PER_W)], idx_v)

    def fire_gathers(s, b):
        for k in range(K):
            pltpu.async_copy(
                table_hbm.at[idx_v.at[s * K + k]],
                bufs[b].at[pl.ds(k * GCH, GCH)],
                sem_g,
            )

    def wait_gathers(b):
        pltpu.make_async_copy(
            table_hbm.at[pl.ds(0, SCH)], bufs[b], sem_g).wait()

    def fire_out(s, b):
        pltpu.async_copy(
            bufs[b], out_hbm.at[pl.ds(rbase + s * SCH, SCH)], sems[b])

    def wait_out(b):
        pltpu.make_async_copy(
            bufs[b], out_hbm.at[pl.ds(rbase, SCH)], sems[b]).wait()

    def step(s, b, fire_ahead=True):
        wait_gathers(b)
        fire_out(s, b)
        if fire_ahead:
            nb = (b + 2) % NBUF
            wait_out(nb)
            fire_gathers(s + 2, nb)

    fire_gathers(0, 0)
    fire_gathers(1, 1)
    wait_gathers(0)
    fire_out(0, 0)
    fire_gathers(2, 2)

    def body(t, carry):
        s = 3 * t + 1
        step(s, 1)
        step(s + 1, 2)
        step(s + 2, 0)
        return carry

    T = (S - 7) // 3
    lax.fori_loop(0, T, body, 0)
    for s in range(3 * T + 1, S):
        step(s, s % NBUF, fire_ahead=(s + 2 < S))
    wait_out((S - 3) % NBUF)
    wait_out((S - 2) % NBUF)
    wait_out((S - 1) % NBUF)


def kernel(token_ids, weight):
    idx = token_ids.reshape(NW * G_PER_W, GCH).astype(jnp.int32)
    wpad = jnp.pad(weight, ((0, 0), (0, DIM)))
    rows = _emb_lookup(idx, wpad)
    return rows[:, :DIM].reshape(SEQS, SLEN, DIM)
